# baseline (speedup 1.0000x reference)
import functools

import jax
import jax.numpy as jnp
from jax import lax
from jax.experimental import pallas as pl
from jax.experimental.pallas import tpu as pltpu
from jax.experimental.pallas import tpu_sc as plsc

B, S = 4096, 200
D = 64
TOTAL = B * S
NC, NS = 2, 16
NW = NC * NS
CHUNK = 128
PER_W = TOTAL // NW
NCHUNK = PER_W // CHUNK


def _sc_gather(idx3, table):
    mesh = plsc.VectorSubcoreMesh(
        core_axis_name="c", subcore_axis_name="s",
        num_cores=NC, num_subcores=NS,
    )

    @functools.partial(
        pl.kernel,
        out_type=jax.ShapeDtypeStruct((TOTAL, D), jnp.float32),
        mesh=mesh,
        compiler_params=pltpu.CompilerParams(use_tc_tiling_on_sc=False),
        scratch_types=[
            pltpu.VMEM((NCHUNK, CHUNK), jnp.int32),
            pltpu.VMEM((CHUNK, D), jnp.float32),
            pltpu.VMEM((CHUNK, D), jnp.float32),
            pltpu.SemaphoreType.DMA,
            pltpu.SemaphoreType.DMA,
        ],
    )
    def k(idx_hbm, table_hbm, out_hbm, idx_v, buf0, buf1, gsem, osem):
        wid = lax.axis_index("s") * NC + lax.axis_index("c")
        base = wid * PER_W
        pltpu.sync_copy(idx_hbm.at[wid], idx_v)

        @pl.loop(0, NCHUNK, step=2)
        def _(j):
            g0 = pltpu.async_copy(table_hbm.at[idx_v.at[j]], buf0, gsem)
            g1 = pltpu.async_copy(table_hbm.at[idx_v.at[j + 1]], buf1, gsem)
            g0.wait()
            s0 = pltpu.async_copy(
                buf0, out_hbm.at[pl.ds(base + j * CHUNK, CHUNK)], osem)
            g1.wait()
            s1 = pltpu.async_copy(
                buf1, out_hbm.at[pl.ds(base + (j + 1) * CHUNK, CHUNK)], osem)
            s0.wait()
            s1.wait()

    return k(idx3, table)


def kernel(index, emb_weight):
    idx3 = index.reshape(NW, NCHUNK, CHUNK)
    out = _sc_gather(idx3, emb_weight)
    return out.reshape(B, S, D)


# speedup vs baseline: 1.0367x; 1.0367x over previous
import functools

import jax
import jax.numpy as jnp
from jax import lax
from jax.experimental import pallas as pl
from jax.experimental.pallas import tpu as pltpu
from jax.experimental.pallas import tpu_sc as plsc

B, S = 4096, 200
D = 64
TOTAL = B * S
NC, NS = 2, 16
NW = NC * NS
CHUNK = 128
PER_W = TOTAL // NW
NCHUNK = PER_W // CHUNK
NBUF = 12
P = 8


def _sc_gather(idx3, table):
    mesh = plsc.VectorSubcoreMesh(
        core_axis_name="c", subcore_axis_name="s",
        num_cores=NC, num_subcores=NS,
    )

    @functools.partial(
        pl.kernel,
        out_type=jax.ShapeDtypeStruct((TOTAL, D), jnp.float32),
        mesh=mesh,
        compiler_params=pltpu.CompilerParams(use_tc_tiling_on_sc=False),
        scratch_types=[
            pltpu.VMEM((NCHUNK, CHUNK), jnp.int32),
            pltpu.VMEM((NBUF, CHUNK, D), jnp.float32),
            pltpu.SemaphoreType.DMA,
            pltpu.SemaphoreType.DMA,
        ],
    )
    def k(idx_hbm, table_hbm, out_hbm, idx_v, bufs, gsem, osem):
        wid = lax.axis_index("s") * NC + lax.axis_index("c")
        base = wid * PER_W
        pltpu.sync_copy(idx_hbm.at[wid], idx_v)

        def fire_gather(c, b):
            pltpu.async_copy(table_hbm.at[idx_v.at[c]], bufs.at[b], gsem)

        def fire_store(c, b):
            pltpu.async_copy(
                bufs.at[b], out_hbm.at[pl.ds(base + c * CHUNK, CHUNK)], osem)

        def wait_gather():
            pltpu.make_async_copy(
                table_hbm.at[pl.ds(0, CHUNK)], bufs.at[0], gsem).wait()

        def wait_store():
            pltpu.make_async_copy(
                bufs.at[0], out_hbm.at[pl.ds(0, CHUNK)], osem).wait()

        for c in range(NBUF):
            fire_gather(c, c)
            if c >= P:
                wait_gather()
                fire_store(c - P, c - P)

        @pl.loop(NBUF, NCHUNK)
        def _(c):
            wait_store()
            fire_gather(c, lax.rem(c, NBUF))
            wait_gather()
            cp = c - P
            fire_store(cp, lax.rem(cp, NBUF))

        for c in range(NCHUNK, NCHUNK + P):
            wait_gather()
            fire_store(c - P, (c - P) % NBUF)
        for _i in range(NBUF):
            wait_store()

    return k(idx3, table)


def kernel(index, emb_weight):
    idx3 = index.reshape(NW, NCHUNK, CHUNK)
    out = _sc_gather(idx3, emb_weight)
    return out.reshape(B, S, D)
